# baseline (speedup 1.0000x reference)
import jax
import jax.numpy as jnp
from jax import lax
from jax.experimental import pallas as pl
from jax.experimental.pallas import tpu as pltpu
from jax.experimental.pallas import tpu_sc as plsc

_DIM = 4096
_E = 64
_K = 8
_SCALE = 2.5
_TOK = 16384
_NW = 32
_TPW = _TOK // _NW
_L = 16


def _logits_body(x_ref, wt_ref, e_ref, ps_ref):
    i = pl.program_id(0)
    logits = jax.lax.dot_general(
        x_ref[...], wt_ref[...], (((1,), (0,)), ((), ())),
        preferred_element_type=jnp.float32) * _SCALE
    mx = jnp.max(logits, axis=1, keepdims=True)
    ee = jnp.exp(logits - mx)
    e_ref[...] = ee
    tot = jnp.sum(ee, axis=1, keepdims=True)
    pp = jnp.sum(ee / tot, axis=0, keepdims=True)

    @pl.when(i == 0)
    def _init():
        ps_ref[...] = jnp.zeros_like(ps_ref)

    ps_ref[...] += pp


def _logits(x, wt, bt=1024):
    return pl.pallas_call(
        _logits_body,
        grid=(_TOK // bt,),
        in_specs=[
            pl.BlockSpec((bt, _DIM), lambda i: (i, 0)),
            pl.BlockSpec((_DIM, _E), lambda i: (0, 0)),
        ],
        out_specs=[
            pl.BlockSpec((bt, _E), lambda i: (i, 0)),
            pl.BlockSpec((1, _E), lambda i: (0, 0)),
        ],
        out_shape=[
            jax.ShapeDtypeStruct((_TOK, _E), jnp.float32),
            jax.ShapeDtypeStruct((1, _E), jnp.float32),
        ],
        compiler_params=pltpu.CompilerParams(
            dimension_semantics=("arbitrary",)),
    )(x, wt)


def _route_body(l_hbm, w_hbm, i_hbm, f_hbm, l_vm, w_vm, i_vm, f_vm):
    cid = lax.axis_index("c")
    sid = lax.axis_index("s")
    wid = sid * 2 + cid
    base = wid * _TPW
    pltpu.sync_copy(l_hbm.at[pl.ds(base, _TPW), :], l_vm)

    lane = lax.iota(jnp.int32, _L)
    m8 = lane < _K
    idx = [lane + j * _L for j in range(4)]
    ones = jnp.ones((_L,), jnp.float32)
    zeros = jnp.zeros((_L,), jnp.float32)
    for j in range(4):
        f_vm[pl.ds(j * _L, _L)] = zeros

    def merge(ka, va, kb, vb):
        ck = jnp.where(m8, ka, jnp.flip(kb))
        cv = jnp.where(m8, va, jnp.flip(vb))
        return plsc.sort_key_val(ck, cv, descending=True)

    @plsc.parallel_loop(0, _TPW, 1, unroll=1)
    def _token(t):
        s = [l_vm[t, pl.ds(j * _L, _L)] for j in range(4)]
        kv = [plsc.sort_key_val(s[j], idx[j], descending=True)
              for j in range(4)]
        ka, va = merge(kv[0][0], kv[0][1], kv[1][0], kv[1][1])
        kb, vb = merge(kv[2][0], kv[2][1], kv[3][0], kv[3][1])
        kt, vt = merge(ka, va, kb, vb)
        plsc.store_compressed(w_vm.at[pl.ds(t * _K, _L)], kt, mask=m8)
        plsc.store_compressed(i_vm.at[pl.ds(t * _K, _L)], vt, mask=m8)
        plsc.addupdate_scatter(f_vm, [vt], ones, mask=m8)

    n = _TPW * _K
    pltpu.sync_copy(w_vm.at[pl.ds(0, n)], w_hbm.at[pl.ds(base * _K, n)])
    pltpu.sync_copy(i_vm.at[pl.ds(0, n)], i_hbm.at[pl.ds(base * _K, n)])
    pltpu.sync_copy(f_vm, f_hbm.at[wid])


def _route(evals):
    mesh = plsc.VectorSubcoreMesh(core_axis_name="c", subcore_axis_name="s")
    return pl.kernel(
        _route_body,
        out_type=(
            jax.ShapeDtypeStruct((_TOK * _K,), jnp.float32),
            jax.ShapeDtypeStruct((_TOK * _K,), jnp.int32),
            jax.ShapeDtypeStruct((_NW, _E), jnp.float32),
        ),
        mesh=mesh,
        scratch_types=[
            pltpu.VMEM((_TPW, _E), jnp.float32),
            pltpu.VMEM((_TPW * _K + _L,), jnp.float32),
            pltpu.VMEM((_TPW * _K + _L,), jnp.int32),
            pltpu.VMEM((_E,), jnp.float32),
        ],
        compiler_params=pltpu.CompilerParams(needs_layout_passes=False),
    )(evals)


_WROWS = _TOK * _K // 128


def _fin_body(wr_ref, f_ref, p_ref, w_ref, o_ref):
    lane = lax.broadcasted_iota(jnp.int32, (128, 16), 0) // _K
    grp = lax.broadcasted_iota(jnp.int32, (128, 16), 1)
    seg = (lane == grp).astype(jnp.float32)
    wr = wr_ref[...]
    sums = jax.lax.dot_general(
        wr, seg, (((1,), (0,)), ((), ())),
        precision=jax.lax.Precision.HIGHEST,
        preferred_element_type=jnp.float32)
    div = jax.lax.dot_general(
        sums, seg, (((1,), (1,)), ((), ())),
        precision=jax.lax.Precision.HIGHEST,
        preferred_element_type=jnp.float32)
    w_ref[...] = wr / div
    fsum = jnp.sum(f_ref[...], axis=0)
    o_ref[0, 0] = jnp.sum(fsum * p_ref[0, :]) * (_E / (_TOK * _TOK))


def _finalize(w_raw, f_part, p_sum):
    return pl.pallas_call(
        _fin_body,
        out_specs=[
            pl.BlockSpec(memory_space=pltpu.VMEM),
            pl.BlockSpec(memory_space=pltpu.SMEM),
        ],
        out_shape=[
            jax.ShapeDtypeStruct((_WROWS, 128), jnp.float32),
            jax.ShapeDtypeStruct((1, 1), jnp.float32),
        ],
    )(w_raw, f_part, p_sum)


def kernel(x, W):
    wt = W.T
    evals, p_sum = _logits(x, wt)
    w_flat, i_flat, f_part = _route(evals)
    w_norm, aux = _finalize(w_flat.reshape(_WROWS, 128), f_part, p_sum)
    return (w_norm.reshape(_TOK, _K),
            i_flat.reshape(_TOK, _K),
            aux[0, 0])


# speedup vs baseline: 1.0045x; 1.0045x over previous
import jax
import jax.numpy as jnp
from jax import lax
from jax.experimental import pallas as pl
from jax.experimental.pallas import tpu as pltpu
from jax.experimental.pallas import tpu_sc as plsc

_DIM = 4096
_E = 64
_K = 8
_SCALE = 2.5
_TOK = 16384
_NW = 32
_TPW = _TOK // _NW
_L = 16


def _logits_body(x_ref, wt_ref, e_ref, ps_ref):
    i = pl.program_id(0)
    logits = jax.lax.dot_general(
        x_ref[...], wt_ref[...], (((1,), (0,)), ((), ())),
        preferred_element_type=jnp.float32) * _SCALE
    mx = jnp.max(logits, axis=1, keepdims=True)
    ee = jnp.exp(logits - mx)
    e_ref[...] = ee
    tot = jnp.sum(ee, axis=1, keepdims=True)
    pp = jnp.sum(ee / tot, axis=0, keepdims=True)

    @pl.when(i == 0)
    def _init():
        ps_ref[...] = jnp.zeros_like(ps_ref)

    ps_ref[...] += pp


def _logits(x, wt, bt=1024):
    return pl.pallas_call(
        _logits_body,
        grid=(_TOK // bt,),
        in_specs=[
            pl.BlockSpec((bt, _DIM), lambda i: (i, 0)),
            pl.BlockSpec((_DIM, _E), lambda i: (0, 0)),
        ],
        out_specs=[
            pl.BlockSpec((bt, _E), lambda i: (i, 0)),
            pl.BlockSpec((1, _E), lambda i: (0, 0)),
        ],
        out_shape=[
            jax.ShapeDtypeStruct((_TOK, _E), jnp.float32),
            jax.ShapeDtypeStruct((1, _E), jnp.float32),
        ],
        compiler_params=pltpu.CompilerParams(
            dimension_semantics=("arbitrary",)),
    )(x, wt)


def _route_body(l_hbm, w_hbm, i_hbm, f_hbm,
                l_vm, w_vm, i_vm, f_vm, sem_in, sem_out):
    cid = lax.axis_index("c")
    sid = lax.axis_index("s")
    wid = sid * 2 + cid
    base = wid * _TPW
    half = _TPW // 2
    pltpu.sync_copy(l_hbm.at[pl.ds(base, half), :],
                    l_vm.at[pl.ds(0, half), :])
    in2 = pltpu.make_async_copy(l_hbm.at[pl.ds(base + half, half), :],
                                l_vm.at[pl.ds(half, half), :], sem_in)
    in2.start()

    lane = lax.iota(jnp.int32, _L)
    m8 = lane < _K
    idx = [lane + j * _L for j in range(4)]
    ones = jnp.ones((_L,), jnp.float32)
    zeros = jnp.zeros((_L,), jnp.float32)
    for j in range(4):
        f_vm[pl.ds(j * _L, _L)] = zeros

    def merge(ka, va, kb, vb):
        ck = jnp.where(m8, ka, jnp.flip(kb))
        cv = jnp.where(m8, va, jnp.flip(vb))
        return plsc.sort_key_val(ck, cv, descending=True)

    def _token(t):
        s = [l_vm[t, pl.ds(j * _L, _L)] for j in range(4)]
        kv = [plsc.sort_key_val(s[j], idx[j], descending=True)
              for j in range(4)]
        ka, va = merge(kv[0][0], kv[0][1], kv[1][0], kv[1][1])
        kb, vb = merge(kv[2][0], kv[2][1], kv[3][0], kv[3][1])
        kt, vt = merge(ka, va, kb, vb)
        plsc.store_compressed(w_vm.at[pl.ds(t * _K, _L)], kt, mask=m8)
        plsc.store_compressed(i_vm.at[pl.ds(t * _K, _L)], vt, mask=m8)
        plsc.addupdate_scatter(f_vm, [vt], ones, mask=m8)

    plsc.parallel_loop(0, half, 1, unroll=2)(_token)

    hn = half * _K
    ow1 = pltpu.make_async_copy(w_vm.at[pl.ds(0, hn)],
                                w_hbm.at[pl.ds(base * _K, hn)], sem_out)
    oi1 = pltpu.make_async_copy(i_vm.at[pl.ds(0, hn)],
                                i_hbm.at[pl.ds(base * _K, hn)], sem_out)
    ow1.start()
    oi1.start()
    in2.wait()
    plsc.parallel_loop(half, _TPW, 1, unroll=2)(_token)

    pltpu.sync_copy(w_vm.at[pl.ds(hn, hn)],
                    w_hbm.at[pl.ds(base * _K + hn, hn)])
    pltpu.sync_copy(i_vm.at[pl.ds(hn, hn)],
                    i_hbm.at[pl.ds(base * _K + hn, hn)])
    pltpu.sync_copy(f_vm, f_hbm.at[wid])
    ow1.wait()
    oi1.wait()


def _route(evals):
    mesh = plsc.VectorSubcoreMesh(core_axis_name="c", subcore_axis_name="s")
    return pl.kernel(
        _route_body,
        out_type=(
            jax.ShapeDtypeStruct((_TOK * _K,), jnp.float32),
            jax.ShapeDtypeStruct((_TOK * _K,), jnp.int32),
            jax.ShapeDtypeStruct((_NW, _E), jnp.float32),
        ),
        mesh=mesh,
        scratch_types=[
            pltpu.VMEM((_TPW, _E), jnp.float32),
            pltpu.VMEM((_TPW * _K + _L,), jnp.float32),
            pltpu.VMEM((_TPW * _K + _L,), jnp.int32),
            pltpu.VMEM((_E,), jnp.float32),
            pltpu.SemaphoreType.DMA,
            pltpu.SemaphoreType.DMA,
        ],
        compiler_params=pltpu.CompilerParams(needs_layout_passes=False),
    )(evals)


_WROWS = _TOK * _K // 128


def _fin_body(wr_ref, f_ref, p_ref, w_ref, o_ref):
    lane = lax.broadcasted_iota(jnp.int32, (128, 16), 0) // _K
    grp = lax.broadcasted_iota(jnp.int32, (128, 16), 1)
    seg = (lane == grp).astype(jnp.float32)
    wr = wr_ref[...]
    sums = jax.lax.dot_general(
        wr, seg, (((1,), (0,)), ((), ())),
        precision=jax.lax.Precision.HIGHEST,
        preferred_element_type=jnp.float32)
    div = jax.lax.dot_general(
        sums, seg, (((1,), (1,)), ((), ())),
        precision=jax.lax.Precision.HIGHEST,
        preferred_element_type=jnp.float32)
    w_ref[...] = wr / div
    fsum = jnp.sum(f_ref[...], axis=0)
    o_ref[0, 0] = jnp.sum(fsum * p_ref[0, :]) * (_E / (_TOK * _TOK))


def _finalize(w_raw, f_part, p_sum):
    return pl.pallas_call(
        _fin_body,
        out_specs=[
            pl.BlockSpec(memory_space=pltpu.VMEM),
            pl.BlockSpec(memory_space=pltpu.SMEM),
        ],
        out_shape=[
            jax.ShapeDtypeStruct((_WROWS, 128), jnp.float32),
            jax.ShapeDtypeStruct((1, 1), jnp.float32),
        ],
    )(w_raw, f_part, p_sum)


def kernel(x, W):
    wt = W.T
    evals, p_sum = _logits(x, wt)
    w_flat, i_flat, f_part = _route(evals)
    w_norm, aux = _finalize(w_flat.reshape(_WROWS, 128), f_part, p_sum)
    return (w_norm.reshape(_TOK, _K),
            i_flat.reshape(_TOK, _K),
            aux[0, 0])


# speedup vs baseline: 1.0302x; 1.0257x over previous
import jax
import jax.numpy as jnp
from jax import lax
from jax.experimental import pallas as pl
from jax.experimental.pallas import tpu as pltpu
from jax.experimental.pallas import tpu_sc as plsc

_DIM = 4096
_E = 64
_K = 8
_SCALE = 2.5
_TOK = 16384
_NW = 32
_TPW = _TOK // _NW
_L = 16


def _logits_body(x_ref, w_ref, e_ref, ps_ref):
    i = pl.program_id(0)
    logits = jax.lax.dot_general(
        x_ref[...], w_ref[...], (((1,), (1,)), ((), ())),
        preferred_element_type=jnp.float32) * _SCALE
    mx = jnp.max(logits, axis=1, keepdims=True)
    ee = jnp.exp(logits - mx)
    e_ref[...] = ee
    tot = jnp.sum(ee, axis=1, keepdims=True)
    pp = jnp.sum(ee / tot, axis=0, keepdims=True)

    @pl.when(i == 0)
    def _init():
        ps_ref[...] = jnp.zeros_like(ps_ref)

    ps_ref[...] += pp


def _logits(x, wt, bt=1024):
    return pl.pallas_call(
        _logits_body,
        grid=(_TOK // bt,),
        in_specs=[
            pl.BlockSpec((bt, _DIM), lambda i: (i, 0)),
            pl.BlockSpec((_E, _DIM), lambda i: (0, 0)),
        ],
        out_specs=[
            pl.BlockSpec((bt, _E), lambda i: (i, 0)),
            pl.BlockSpec((1, _E), lambda i: (0, 0)),
        ],
        out_shape=[
            jax.ShapeDtypeStruct((_TOK, _E), jnp.float32),
            jax.ShapeDtypeStruct((1, _E), jnp.float32),
        ],
        compiler_params=pltpu.CompilerParams(
            dimension_semantics=("arbitrary",)),
    )(x, wt)


def _route_body(l_hbm, w_hbm, i_hbm, f_hbm,
                l_vm, w_vm, i_vm, f_vm, sem_in, sem_out):
    cid = lax.axis_index("c")
    sid = lax.axis_index("s")
    wid = sid * 2 + cid
    base = wid * _TPW
    half = _TPW // 2
    pltpu.sync_copy(l_hbm.at[pl.ds(base, half), :],
                    l_vm.at[pl.ds(0, half), :])
    in2 = pltpu.make_async_copy(l_hbm.at[pl.ds(base + half, half), :],
                                l_vm.at[pl.ds(half, half), :], sem_in)
    in2.start()

    lane = lax.iota(jnp.int32, _L)
    m8 = lane < _K
    idx = [lane + j * _L for j in range(4)]
    ones = jnp.ones((_L,), jnp.float32)
    zeros = jnp.zeros((_L,), jnp.float32)
    for j in range(4):
        f_vm[pl.ds(j * _L, _L)] = zeros

    def merge(ka, va, kb, vb):
        ck = jnp.where(m8, ka, jnp.flip(kb))
        cv = jnp.where(m8, va, jnp.flip(vb))
        return plsc.sort_key_val(ck, cv, descending=True)

    def _token(t):
        s = [l_vm[t, pl.ds(j * _L, _L)] for j in range(4)]
        kv = [plsc.sort_key_val(s[j], idx[j], descending=True)
              for j in range(4)]
        ka, va = merge(kv[0][0], kv[0][1], kv[1][0], kv[1][1])
        kb, vb = merge(kv[2][0], kv[2][1], kv[3][0], kv[3][1])
        kt, vt = merge(ka, va, kb, vb)
        plsc.store_compressed(w_vm.at[pl.ds(t * _K, _L)], kt, mask=m8)
        plsc.store_compressed(i_vm.at[pl.ds(t * _K, _L)], vt, mask=m8)
        plsc.addupdate_scatter(f_vm, [vt], ones, mask=m8)

    plsc.parallel_loop(0, half, 1, unroll=2)(_token)

    hn = half * _K
    ow1 = pltpu.make_async_copy(w_vm.at[pl.ds(0, hn)],
                                w_hbm.at[pl.ds(base * _K, hn)], sem_out)
    oi1 = pltpu.make_async_copy(i_vm.at[pl.ds(0, hn)],
                                i_hbm.at[pl.ds(base * _K, hn)], sem_out)
    ow1.start()
    oi1.start()
    in2.wait()
    plsc.parallel_loop(half, _TPW, 1, unroll=2)(_token)

    pltpu.sync_copy(w_vm.at[pl.ds(hn, hn)],
                    w_hbm.at[pl.ds(base * _K + hn, hn)])
    pltpu.sync_copy(i_vm.at[pl.ds(hn, hn)],
                    i_hbm.at[pl.ds(base * _K + hn, hn)])
    pltpu.sync_copy(f_vm, f_hbm.at[wid])
    ow1.wait()
    oi1.wait()


def _route(evals):
    mesh = plsc.VectorSubcoreMesh(core_axis_name="c", subcore_axis_name="s")
    return pl.kernel(
        _route_body,
        out_type=(
            jax.ShapeDtypeStruct((_TOK * _K,), jnp.float32),
            jax.ShapeDtypeStruct((_TOK * _K,), jnp.int32),
            jax.ShapeDtypeStruct((_NW, _E), jnp.float32),
        ),
        mesh=mesh,
        scratch_types=[
            pltpu.VMEM((_TPW, _E), jnp.float32),
            pltpu.VMEM((_TPW * _K + _L,), jnp.float32),
            pltpu.VMEM((_TPW * _K + _L,), jnp.int32),
            pltpu.VMEM((_E,), jnp.float32),
            pltpu.SemaphoreType.DMA,
            pltpu.SemaphoreType.DMA,
        ],
        compiler_params=pltpu.CompilerParams(needs_layout_passes=False),
    )(evals)


_WROWS = _TOK * _K // 128


def _fin_body(wr_ref, f_ref, p_ref, w_ref, o_ref):
    lane = lax.broadcasted_iota(jnp.int32, (128, 16), 0) // _K
    grp = lax.broadcasted_iota(jnp.int32, (128, 16), 1)
    seg = (lane == grp).astype(jnp.float32)
    wr = wr_ref[...]
    sums = jax.lax.dot_general(
        wr, seg, (((1,), (0,)), ((), ())),
        precision=jax.lax.Precision.HIGHEST,
        preferred_element_type=jnp.float32)
    div = jax.lax.dot_general(
        sums, seg, (((1,), (1,)), ((), ())),
        precision=jax.lax.Precision.HIGHEST,
        preferred_element_type=jnp.float32)
    w_ref[...] = wr / div
    fsum = jnp.sum(f_ref[...], axis=0)
    o_ref[0, 0] = jnp.sum(fsum * p_ref[0, :]) * (_E / (_TOK * _TOK))


def _finalize(w_raw, f_part, p_sum):
    return pl.pallas_call(
        _fin_body,
        out_specs=[
            pl.BlockSpec(memory_space=pltpu.VMEM),
            pl.BlockSpec(memory_space=pltpu.SMEM),
        ],
        out_shape=[
            jax.ShapeDtypeStruct((_WROWS, 128), jnp.float32),
            jax.ShapeDtypeStruct((1, 1), jnp.float32),
        ],
    )(w_raw, f_part, p_sum)


def kernel(x, W):
    evals, p_sum = _logits(x, W)
    w_flat, i_flat, f_part = _route(evals)
    w_norm, aux = _finalize(w_flat.reshape(_WROWS, 128), f_part, p_sum)
    return (w_norm.reshape(_TOK, _K),
            i_flat.reshape(_TOK, _K),
            aux[0, 0])
